# baseline (speedup 1.0000x reference)
import functools

import jax
import jax.numpy as jnp
from jax import lax
from jax.experimental import pallas as pl
from jax.experimental.pallas import tpu as pltpu
from jax.experimental.pallas import tpu_sc as plsc

NUM_EFFECTS = 100
EMBED_DIM = 128
BATCH = 16384
_SC_ROWS = 8192
_TC_ROWS = BATCH - _SC_ROWS

_INFO = plsc.get_sparse_core_info()
_NC = _INFO.num_cores
_NS = _INFO.num_subcores
_NW = _NC * _NS
_B_PER_W = _SC_ROWS // _NW
_CHUNK = 128
_NCHUNK = _B_PER_W // _CHUNK

_mesh = plsc.VectorSubcoreMesh(core_axis_name="c", subcore_axis_name="s")


@functools.partial(
    pl.kernel,
    mesh=_mesh,
    out_type=jax.ShapeDtypeStruct((_SC_ROWS, EMBED_DIM), jnp.float32),
    scratch_types=[
        pltpu.VMEM((_NCHUNK, _CHUNK), jnp.int32),
        pltpu.VMEM((_B_PER_W, EMBED_DIM), jnp.float32),
        pltpu.VMEM_SHARED((NUM_EFFECTS, EMBED_DIM), jnp.float32),
    ]
    + [pltpu.SemaphoreType.DMA] * _NCHUNK
    + [pltpu.SemaphoreType.DMA],
)
def _gather_kernel(idx_hbm, table_hbm, out_hbm, idx_v, rows_v, table_sh, *sems):
    gsems, ssem = sems[:_NCHUNK], sems[_NCHUNK]
    s = lax.axis_index("s")
    wid = s * _NC + lax.axis_index("c")
    base = wid * _B_PER_W

    @pl.when(s == 0)
    def _load_table():
        pltpu.sync_copy(table_hbm, table_sh)

    pltpu.sync_copy(idx_hbm.at[wid], idx_v)
    plsc.subcore_barrier()

    gathers = [
        pltpu.async_copy(
            table_sh.at[idx_v.at[j]],
            rows_v.at[pl.ds(j * _CHUNK, _CHUNK)],
            gsems[j],
        )
        for j in range(_NCHUNK)
    ]
    stores = []
    for j in range(_NCHUNK):
        gathers[j].wait()
        stores.append(
            pltpu.async_copy(
                rows_v.at[pl.ds(j * _CHUNK, _CHUNK)],
                out_hbm.at[pl.ds(base + j * _CHUNK, _CHUNK)],
                ssem,
            )
        )
    for st in stores:
        st.wait()


_TC_BLK = 1024
_TC_OFF = _SC_ROWS // _TC_BLK


def _onehot_body(idx_ref, tab_ref, out_ref):
    tab = tab_ref[...]
    rows = lax.broadcasted_iota(jnp.int32, (128, 128), 0)
    for i in range(_TC_BLK // 128):
        idx_lane = idx_ref[0, i]
        m = (rows == idx_lane[None, :]).astype(jnp.float32)
        out_ref[pl.ds(i * 128, 128), :] = lax.dot_general(
            m, tab, (((0,), (0,)), ((), ())),
            preferred_element_type=jnp.float32,
        )


_onehot_call = pl.pallas_call(
    _onehot_body,
    grid=(_TC_ROWS // _TC_BLK,),
    in_specs=[
        pl.BlockSpec((1, _TC_BLK // 128, 128), lambda i: (_TC_OFF + i, 0, 0)),
        pl.BlockSpec((128, 128), lambda i: (0, 0)),
    ],
    out_specs=pl.BlockSpec((_TC_BLK, 128), lambda i: (_TC_OFF + i, 0)),
    out_shape=jax.ShapeDtypeStruct((BATCH, EMBED_DIM), jnp.float32),
)


def kernel(effect_id, embedding):
    idx_sc = effect_id[:_SC_ROWS].reshape(_NW, _NCHUNK, _CHUNK)
    sc_part = _gather_kernel(idx_sc, embedding)
    table_pad = jnp.pad(embedding, ((0, 128 - NUM_EFFECTS), (0, 0)))
    idx_sq = effect_id.reshape(BATCH // _TC_BLK, _TC_BLK // 128, 128)
    tc_full = _onehot_call(idx_sq, table_pad)
    return lax.dynamic_update_slice(tc_full, sc_part, (0, 0))


# speedup vs baseline: 1.1884x; 1.1884x over previous
import functools

import jax
import jax.numpy as jnp
from jax import lax
from jax.experimental import pallas as pl
from jax.experimental.pallas import tpu as pltpu
from jax.experimental.pallas import tpu_sc as plsc

NUM_EFFECTS = 100
EMBED_DIM = 128
BATCH = 16384

_INFO = plsc.get_sparse_core_info()
_NC = _INFO.num_cores
_NS = _INFO.num_subcores
_NW = _NC * _NS
_B_PER_W = BATCH // _NW
_CHUNK = 128
_NCHUNK = _B_PER_W // _CHUNK

_mesh = plsc.VectorSubcoreMesh(core_axis_name="c", subcore_axis_name="s")


@functools.partial(
    pl.kernel,
    mesh=_mesh,
    out_type=jax.ShapeDtypeStruct((BATCH, EMBED_DIM), jnp.float32),
    scratch_types=[
        pltpu.VMEM((_NCHUNK, _CHUNK), jnp.int32),
        pltpu.VMEM((_B_PER_W, EMBED_DIM), jnp.float32),
        pltpu.VMEM_SHARED((NUM_EFFECTS, EMBED_DIM), jnp.float32),
    ]
    + [pltpu.SemaphoreType.DMA] * _NCHUNK
    + [pltpu.SemaphoreType.DMA, pltpu.SemaphoreType.DMA],
)
def _gather_kernel(idx_hbm, table_hbm, out_hbm, idx_v, rows_v, table_sh, *sems):
    gsems, ssem, isem = sems[:_NCHUNK], sems[_NCHUNK], sems[_NCHUNK + 1]
    s = lax.axis_index("s")
    wid = s * _NC + lax.axis_index("c")
    base = wid * _B_PER_W

    idx_cp = pltpu.async_copy(idx_hbm.at[wid], idx_v, isem)

    @pl.when(s == 0)
    def _load_table():
        pltpu.sync_copy(table_hbm, table_sh)

    plsc.subcore_barrier()
    idx_cp.wait()

    gathers = [
        pltpu.async_copy(
            table_sh.at[idx_v.at[j]],
            rows_v.at[pl.ds(j * _CHUNK, _CHUNK)],
            gsems[j],
        )
        for j in range(_NCHUNK)
    ]
    stores = []
    for j in range(_NCHUNK):
        gathers[j].wait()
        stores.append(
            pltpu.async_copy(
                rows_v.at[pl.ds(j * _CHUNK, _CHUNK)],
                out_hbm.at[pl.ds(base + j * _CHUNK, _CHUNK)],
                ssem,
            )
        )
    for st in stores:
        st.wait()


def kernel(effect_id, embedding):
    idx = effect_id.reshape(_NW, _NCHUNK, _CHUNK)
    return _gather_kernel(idx, embedding)
